# baseline (speedup 1.0000x reference)
import functools

import jax
import jax.numpy as jnp
from jax import lax
from jax.experimental import pallas as pl
from jax.experimental.pallas import tpu as pltpu
from jax.experimental.pallas import tpu_sc as plsc

D_MODEL = 64
SCALE = 8.0
NUM_WORKERS = 32
CHUNK = 800


def _emb_body(idx_hbm, table_hbm, out_hbm, idx_v, rows_v, sem, *, b_per_w):
    wid = lax.axis_index("s") * 2 + lax.axis_index("c")
    base = wid * b_per_w
    n_chunks = b_per_w // CHUNK

    def chunk_body(g, carry):
        off = base + g * CHUNK
        pltpu.sync_copy(idx_hbm.at[pl.ds(off, CHUNK)], idx_v)
        pltpu.async_copy(table_hbm.at[idx_v], rows_v, sem).wait()

        def row_body(r, c):
            for j in range(D_MODEL // 16):
                sl = pl.ds(j * 16, 16)
                rows_v[r, sl] = rows_v[r, sl] * SCALE
            return c

        lax.fori_loop(0, CHUNK, row_body, 0)
        pltpu.sync_copy(rows_v, out_hbm.at[pl.ds(off, CHUNK)])
        return carry

    lax.fori_loop(0, n_chunks, chunk_body, 0)


def kernel(x, emb_weight):
    orig_shape = x.shape
    b_total = x.size
    b_per_w = b_total // NUM_WORKERS
    xf = x.reshape(b_total).astype(jnp.int32)

    mesh = plsc.VectorSubcoreMesh(core_axis_name="c", subcore_axis_name="s")

    emb = functools.partial(
        pl.kernel,
        mesh=mesh,
        out_type=jax.ShapeDtypeStruct((b_total, D_MODEL), jnp.float32),
        scratch_types=[
            pltpu.VMEM((CHUNK,), jnp.int32),
            pltpu.VMEM((CHUNK, D_MODEL), jnp.float32),
            pltpu.SemaphoreType.DMA,
        ],
        compiler_params=pltpu.CompilerParams(use_tc_tiling_on_sc=False),
    )(functools.partial(_emb_body, b_per_w=b_per_w))

    out = emb(xf, emb_weight)
    return out.reshape(orig_shape + (D_MODEL,))


# speedup vs baseline: 1.1169x; 1.1169x over previous
import functools

import jax
import jax.numpy as jnp
from jax import lax
from jax.experimental import pallas as pl
from jax.experimental.pallas import tpu as pltpu
from jax.experimental.pallas import tpu_sc as plsc

D_MODEL = 64
SCALE = 8.0
NUM_WORKERS = 32
CHUNK = 800
ROW_UNROLL = 4


def _emb_body(idx_hbm, table_hbm, out_hbm, idx_all, rows0, rows1,
              gsem0, gsem1, ssem0, ssem1, *, b_per_w):
    wid = lax.axis_index("s") * 2 + lax.axis_index("c")
    base = wid * b_per_w
    n_chunks = b_per_w // CHUNK
    rows = (rows0, rows1)
    gsem = (gsem0, gsem1)
    ssem = (ssem0, ssem1)

    pltpu.sync_copy(idx_hbm.at[pl.ds(base, b_per_w)], idx_all)

    def start_gather(g, b):
        pltpu.async_copy(
            table_hbm.at[idx_all.at[pl.ds(g * CHUNK, CHUNK)]], rows[b], gsem[b]
        )

    def scale_buf(b):
        def row_body(r4, c):
            r0 = r4 * ROW_UNROLL
            for dr in range(ROW_UNROLL):
                for j in range(D_MODEL // 16):
                    sl = pl.ds(j * 16, 16)
                    rows[b][r0 + dr, sl] = rows[b][r0 + dr, sl] * SCALE
            return c
        lax.fori_loop(0, CHUNK // ROW_UNROLL, row_body, 0)

    start_gather(0, 0)

    def pair_body(p, c):
        for b in range(2):
            g = 2 * p + b
            @pl.when(g + 1 < n_chunks)
            def _():
                @pl.when(g + 1 >= 2)
                def _():
                    pltpu.make_async_copy(
                        rows[1 - b], out_hbm.at[pl.ds(0, CHUNK)], ssem[1 - b]
                    ).wait()
                start_gather(g + 1, 1 - b)
            pltpu.make_async_copy(
                table_hbm.at[idx_all.at[pl.ds(0, CHUNK)]], rows[b], gsem[b]
            ).wait()
            scale_buf(b)
            pltpu.async_copy(
                rows[b], out_hbm.at[pl.ds(base + g * CHUNK, CHUNK)], ssem[b]
            )
        return c

    lax.fori_loop(0, n_chunks // 2, pair_body, 0)
    for b in range(2):
        pltpu.make_async_copy(
            rows[b], out_hbm.at[pl.ds(0, CHUNK)], ssem[b]
        ).wait()


def kernel(x, emb_weight):
    orig_shape = x.shape
    b_total = x.size
    b_per_w = b_total // NUM_WORKERS
    xf = x.reshape(b_total).astype(jnp.int32)

    mesh = plsc.VectorSubcoreMesh(core_axis_name="c", subcore_axis_name="s")

    emb = functools.partial(
        pl.kernel,
        mesh=mesh,
        out_type=jax.ShapeDtypeStruct((b_total, D_MODEL), jnp.float32),
        scratch_types=[
            pltpu.VMEM((b_per_w,), jnp.int32),
            pltpu.VMEM((CHUNK, D_MODEL), jnp.float32),
            pltpu.VMEM((CHUNK, D_MODEL), jnp.float32),
            pltpu.SemaphoreType.DMA,
            pltpu.SemaphoreType.DMA,
            pltpu.SemaphoreType.DMA,
            pltpu.SemaphoreType.DMA,
        ],
        compiler_params=pltpu.CompilerParams(use_tc_tiling_on_sc=False),
    )(functools.partial(_emb_body, b_per_w=b_per_w))

    out = emb(xf, emb_weight)
    return out.reshape(orig_shape + (D_MODEL,))
